# baseline (speedup 1.0000x reference)
import functools

import jax
import jax.numpy as jnp
from jax import lax
from jax.experimental import pallas as pl
from jax.experimental.pallas import tpu as pltpu
from jax.experimental.pallas import tpu_sc as plsc

_SB = 2


def _make_sc_embed(B, S, V, D, NC, NS):
    NW = NC * NS
    BT = B // NW
    n_tok = BT * S
    n_batches = S // _SB
    NI = n_batches // 2
    DT, DI = D // 8, 8
    mesh = plsc.VectorSubcoreMesh(core_axis_name="c", subcore_axis_name="s")

    @functools.partial(
        pl.kernel,
        mesh=mesh,
        out_type=jax.ShapeDtypeStruct((S, DT, NW, DI, BT), jnp.float32),
        compiler_params=pltpu.CompilerParams(
            use_tc_tiling_on_sc=False, needs_layout_passes=False
        ),
        scratch_types=[
            pltpu.VMEM((n_tok,), jnp.int32),
            pltpu.VMEM((_SB * BT, D), jnp.float32),
            pltpu.VMEM((_SB * BT, D), jnp.float32),
            pltpu.VMEM((DT, _SB, DI, BT), jnp.float32),
            pltpu.VMEM((DT, _SB, DI, BT), jnp.float32),
            pltpu.VMEM((_SB * BT,), jnp.int32),
            pltpu.VMEM((_SB * BT,), jnp.int32),
            pltpu.VMEM((BT, D + 1), jnp.float32),
            pltpu.VMEM((BT, D + 1), jnp.float32),
            pltpu.SemaphoreType.DMA,
            pltpu.SemaphoreType.DMA,
            pltpu.SemaphoreType.DMA,
            pltpu.SemaphoreType.DMA,
        ],
    )
    def emb(idx_hbm, table_hbm, out_hbm, idx_v, rows0, rows1, tv0, tv1,
            gi0, gi1, st0, st1, g0, g1, o0, o1):
        wid = lax.axis_index("s") * NC + lax.axis_index("c")
        pltpu.sync_copy(idx_hbm.at[pl.ds(wid * n_tok, n_tok)], idx_v)

        rows = [rows0, rows1]
        tv = [tv0, tv1]
        gidx = [gi0, gi1]
        gsem = [g0, g1]
        osem = [o0, o1]

        iota = lax.iota(jnp.int32, 16)
        iota_s = iota * S

        def build_gidx(p, k):
            for sb in range(_SB):
                s = k * _SB + sb
                for b0 in range(0, BT, 16):
                    rowv = iota_s + (b0 * S + s)
                    gidx[p][pl.ds(sb * BT + b0, 16)] = plsc.load_gather(
                        idx_v, [rowv]
                    )

        def start_gather(p):
            return pltpu.async_copy(table_hbm.at[gidx[p]], rows[p], gsem[p])

        def permute(p):
            stages = [st0, st1]
            for sb in range(_SB):
                st = stages[sb % 2]
                base = sb * BT
                for t in range(BT):
                    st[t, pl.ds(0, 16)] = rows[p][base + t, pl.ds(0, 16)]
                    st[t, pl.ds(16, 16)] = rows[p][base + t, pl.ds(16, 16)]
                for d in range(D):
                    colv = jnp.full((16,), d, jnp.int32)
                    for b0 in range(0, BT, 16):
                        tv[p][d // 8, sb, d % 8, pl.ds(b0, 16)] = (
                            plsc.load_gather(st, [iota + b0, colv])
                        )

        def start_out(p, k):
            s0 = k * _SB
            for dt in range(DT):
                pltpu.async_copy(
                    tv[p].at[dt], out_hbm.at[pl.ds(s0, _SB), dt, wid], osem[p]
                )

        def wait_out(p):
            for _ in range(DT):
                pltpu.make_async_copy(
                    tv[p].at[0], out_hbm.at[pl.ds(0, _SB), 0, wid], osem[p]
                ).wait()

        def wait_gather(p):
            pltpu.make_async_copy(
                table_hbm.at[gidx[p]], rows[p], gsem[p]
            ).wait()

        build_gidx(0, 0)
        start_gather(0)
        build_gidx(1, 1)
        start_gather(1)

        def step(i, p):
            k = 2 * i + p

            @pl.when(i >= 1)
            def _():
                wait_out(p)

            wait_gather(p)
            permute(p)
            start_out(p, k)

            @pl.when(i < NI - 1)
            def _():
                build_gidx(p, k + 2)
                start_gather(p)

        def body(i, carry):
            step(i, 0)
            step(i, 1)
            return carry

        lax.fori_loop(0, NI, body, 0)
        wait_out(0)
        wait_out(1)

    return emb


def kernel(token_ids, table):
    B, S = token_ids.shape
    V, D = table.shape
    idx = token_ids.reshape(B * S).astype(jnp.int32)
    info = plsc.get_sparse_core_info()
    NC, NS = info.num_cores, info.num_subcores
    emb = _make_sc_embed(B, S, V, D, NC, NS)
    out5 = emb(idx, table)
    return out5.transpose(2, 4, 0, 1, 3).reshape(B, S, D)


# speedup vs baseline: 1.1318x; 1.1318x over previous
import functools

import jax
import jax.numpy as jnp
from jax import lax
from jax.experimental import pallas as pl
from jax.experimental.pallas import tpu as pltpu
from jax.experimental.pallas import tpu_sc as plsc

_SB = 4


def _make_sc_embed(B, S, V, D, NC, NS):
    NW = NC * NS
    BT = B // NW
    ST = S // 8
    n_batches = S // _SB
    NI = n_batches // 2
    CH = _SB * BT
    DT, DI = D // 8, 8
    mesh = plsc.VectorSubcoreMesh(core_axis_name="c", subcore_axis_name="s")

    @functools.partial(
        pl.kernel,
        mesh=mesh,
        out_type=jax.ShapeDtypeStruct((S, DT, NW, DI, BT), jnp.float32),
        compiler_params=pltpu.CompilerParams(
            use_tc_tiling_on_sc=False, needs_layout_passes=False
        ),
        scratch_types=[
            pltpu.VMEM((ST, 8 * BT), jnp.int32),
            pltpu.VMEM((CH, D), jnp.float32),
            pltpu.VMEM((CH, D), jnp.float32),
            pltpu.VMEM((DT, _SB, DI, BT), jnp.float32),
            pltpu.VMEM((DT, _SB, DI, BT), jnp.float32),
            pltpu.VMEM((16, D + 1), jnp.float32),
            pltpu.SemaphoreType.DMA,
            pltpu.SemaphoreType.DMA,
            pltpu.SemaphoreType.DMA,
            pltpu.SemaphoreType.DMA,
        ],
    )
    def emb(idx_hbm, table_hbm, out_hbm, idx_v, rows0, rows1, tv0, tv1,
            stage, g0, g1, o0, o1):
        wid = lax.axis_index("s") * NC + lax.axis_index("c")
        pltpu.sync_copy(idx_hbm.at[:, wid], idx_v)

        rows = [rows0, rows1]
        tv = [tv0, tv1]
        gsem = [g0, g1]
        osem = [o0, o1]

        iota = lax.iota(jnp.int32, 16)

        def idx_ref(p, st):
            return idx_v.at[st, pl.ds(p * CH, CH)]

        def start_gather(p, st):
            return pltpu.async_copy(
                table_hbm.at[idx_ref(p, st)], rows[p], gsem[p]
            )

        def wait_gather(p, st):
            pltpu.make_async_copy(
                table_hbm.at[idx_ref(p, st)], rows[p], gsem[p]
            ).wait()

        def permute(p):
            for sb in range(_SB):
                for b0 in range(0, BT, 16):
                    base = sb * BT + b0
                    for t in range(16):
                        stage[t, pl.ds(0, 16)] = rows[p][base + t, pl.ds(0, 16)]
                        stage[t, pl.ds(16, 16)] = rows[p][base + t, pl.ds(16, 16)]
                    for d in range(D):
                        colv = jnp.full((16,), d, jnp.int32)
                        tv[p][d // 8, sb, d % 8, pl.ds(b0, 16)] = (
                            plsc.load_gather(stage, [iota, colv])
                        )

        def start_out(p, k):
            s0 = k * _SB
            for dt in range(DT):
                pltpu.async_copy(
                    tv[p].at[dt], out_hbm.at[pl.ds(s0, _SB), dt, wid], osem[p]
                )

        def wait_out(p):
            for _ in range(DT):
                pltpu.make_async_copy(
                    tv[p].at[0], out_hbm.at[pl.ds(0, _SB), 0, wid], osem[p]
                ).wait()

        start_gather(0, 0)
        start_gather(1, 0)

        def step(i, p):
            k = 2 * i + p

            @pl.when(i >= 1)
            def _():
                wait_out(p)

            wait_gather(p, i)
            permute(p)
            start_out(p, k)

            @pl.when(i < NI - 1)
            def _():
                start_gather(p, i + 1)

        def body(i, carry):
            step(i, 0)
            step(i, 1)
            return carry

        lax.fori_loop(0, NI, body, 0)
        wait_out(0)
        wait_out(1)

    return emb


def kernel(token_ids, table):
    B, S = token_ids.shape
    V, D = table.shape
    idx4 = (
        token_ids.astype(jnp.int32)
        .T.reshape(S // 8, 8, B // 128, 128)
        .transpose(0, 2, 1, 3)
        .reshape(S // 8, B // 128, 8 * 128)
    )
    info = plsc.get_sparse_core_info()
    NC, NS = info.num_cores, info.num_subcores
    emb = _make_sc_embed(B, S, V, D, NC, NS)
    out5 = emb(idx4, table)
    return out5.transpose(2, 4, 0, 1, 3).reshape(B, S, D)
